# baseline (speedup 1.0000x reference)
import jax
import jax.numpy as jnp
from jax.experimental import pallas as pl


_BLOCK_T = 512


def _add_kernel(x_ref, pos_ref, o_ref):
    o_ref[...] = x_ref[...] + pos_ref[...]


def kernel(X, pos_table):
    N, T, D = X.shape
    bt = min(_BLOCK_T, T)
    grid = (T // bt,)
    return pl.pallas_call(
        _add_kernel,
        grid=grid,
        in_specs=[
            pl.BlockSpec((N, bt, D), lambda t: (0, t, 0)),
            pl.BlockSpec((bt, D), lambda t: (t, 0)),
        ],
        out_specs=pl.BlockSpec((N, bt, D), lambda t: (0, t, 0)),
        out_shape=jax.ShapeDtypeStruct((N, T, D), X.dtype),
    )(X, pos_table)
